# baseline (speedup 1.0000x reference)
import functools

import jax
import jax.numpy as jnp
from jax import lax
from jax.experimental import pallas as pl
from jax.experimental.pallas import tpu as pltpu
from jax.experimental.pallas import tpu_sc as plsc

B = 1024
VOCAB = 100000
K = 64
RK = 7
TEMP = 10.0

BB = 128
SBB = 256
SVB = 8192
NW = 32
CH = (B * K) // (NW * 128)

_f32 = jnp.float32


def _mm_t(x, w):
    return lax.dot_general(x, w, (((1,), (1,)), ((), ())),
                           preferred_element_type=_f32)


def _meta_body(tgt_ref, dist_ref, w1k_ref, b1k_ref, w2k_ref, b2k_ref,
               w1l_ref, b1l_ref, w2l_ref, b2l_ref,
               scale_ref, flat_ref, group_ref):
    tgt = tgt_ref[...]
    dist = dist_ref[...]

    eqf = (tgt[:, :, None] == tgt[:, None, :]).astype(_f32)
    ii = lax.broadcasted_iota(jnp.int32, (K, K), 0)
    jj = lax.broadcasted_iota(jnp.int32, (K, K), 1)
    seen = jnp.sum(eqf * (jj < ii).astype(_f32)[None], axis=-1)
    novel = jnp.where((tgt != 0) & (seen == 0.0), 1.0, 0.0).astype(_f32)
    counts = jnp.dot(novel, (ii <= jj).astype(_f32),
                     preferred_element_type=_f32)

    net_in = jnp.concatenate([dist, counts], axis=-1)
    hk = jnp.tanh(_mm_t(net_in, w1k_ref[...]) + b1k_ref[...][None, :])
    lk = _mm_t(hk, w2k_ref[...]) + b2k_ref[...][None, :]
    mx = jnp.max(lk, axis=-1, keepdims=True)
    ek = jnp.exp(lk - mx)
    kp = ek / jnp.sum(ek, axis=-1, keepdims=True)

    hl = jnp.tanh(_mm_t(net_in, w1l_ref[...]) + b1l_ref[...][None, :])
    ll = jnp.sum(hl * w2l_ref[...], axis=-1, keepdims=True) + b2l_ref[0]
    klam = jnp.minimum(jax.nn.sigmoid(ll), 0.99)

    ik = lax.broadcasted_iota(jnp.int32, (BB, K), 1)
    spare = jnp.zeros((BB, K), _f32)
    for r in range(RK):
        m = jnp.where(ik < (1 << r), 1.0, 1000.0).astype(_f32)
        logits = -(dist * m) / TEMP
        mxr = jnp.max(logits, axis=-1, keepdims=True)
        er = jnp.exp(logits - mxr)
        w = er / jnp.sum(er, axis=-1, keepdims=True)
        spare = spare + kp[:, r:r + 1] * w
    spare = klam * spare

    group = jnp.sum(eqf * spare[:, None, :], axis=-1)

    row = pl.program_id(0) * BB + lax.broadcasted_iota(jnp.int32, (BB, K), 0)
    flat_ref[...] = row * VOCAB + tgt
    group_ref[...] = group
    scale_ref[...] = 1.0 - klam


def _meta_call(tgt, dist, w1k, b1k, w2k, b2k, w1l, b1l, w2l, b2l):
    full = lambda a: pl.BlockSpec(a.shape, lambda i: (0,) * a.ndim)
    return pl.pallas_call(
        _meta_body,
        grid=(B // BB,),
        in_specs=[
            pl.BlockSpec((BB, K), lambda i: (i, 0)),
            pl.BlockSpec((BB, K), lambda i: (i, 0)),
            full(w1k), full(b1k), full(w2k), full(b2k),
            full(w1l), full(b1l), full(w2l),
            pl.BlockSpec(memory_space=pltpu.SMEM),
        ],
        out_specs=[
            pl.BlockSpec((BB, 1), lambda i: (i, 0)),
            pl.BlockSpec((BB, K), lambda i: (i, 0)),
            pl.BlockSpec((BB, K), lambda i: (i, 0)),
        ],
        out_shape=[
            jax.ShapeDtypeStruct((B, 1), _f32),
            jax.ShapeDtypeStruct((B, K), jnp.int32),
            jax.ShapeDtypeStruct((B, K), _f32),
        ],
    )(tgt, dist, w1k, b1k, w2k, b2k, w1l, b1l, w2l, b2l)


def _scale_body(nmt_ref, scale_ref, out_ref):
    out_ref[...] = nmt_ref[...] * scale_ref[...]


def _scale_call(nmt_prob, scale):
    nv = pl.cdiv(VOCAB, SVB)
    return pl.pallas_call(
        _scale_body,
        grid=(B // SBB, nv),
        in_specs=[
            pl.BlockSpec((SBB, SVB), lambda i, j: (i, j)),
            pl.BlockSpec((SBB, 1), lambda i, j: (i, 0)),
        ],
        out_specs=pl.BlockSpec((SBB, SVB), lambda i, j: (i, j)),
        out_shape=jax.ShapeDtypeStruct((B, VOCAB), _f32),
    )(nmt_prob, scale)


@functools.cache
def _sc_scatter_fn():
    mesh = plsc.VectorSubcoreMesh(core_axis_name="c", subcore_axis_name="s")

    @functools.partial(
        pl.kernel,
        out_type=(),
        mesh=mesh,
        scratch_types=[
            pltpu.VMEM((CH, 128), jnp.int32),
            pltpu.VMEM((CH, 128), _f32),
            pltpu.VMEM((CH, 128), _f32),
            pltpu.SemaphoreType.DMA,
        ],
    )
    def _sc_scatter(idx_hbm, add_hbm, out_ref, idx_v, add_v, val_v, sem):
        wid = lax.axis_index("s") * 2 + lax.axis_index("c")
        base = wid * CH
        pltpu.sync_copy(idx_hbm.at[pl.ds(base, CH)], idx_v)
        pltpu.sync_copy(add_hbm.at[pl.ds(base, CH)], add_v)
        gathers = [pltpu.async_copy(out_ref.at[idx_v.at[j]], val_v.at[j], sem)
                   for j in range(CH)]
        for g in gathers:
            g.wait()
        for j in range(CH):
            for i in range(8):
                s = pl.ds(i * 16, 16)
                val_v[j, s] = val_v[j, s] + add_v[j, s]
        scatters = [pltpu.async_copy(val_v.at[j], out_ref.at[idx_v.at[j]], sem)
                    for j in range(CH)]
        for sc in scatters:
            sc.wait()

    return _sc_scatter


def kernel(nmt_prob, knn_tgt, knn_dist, knn_alpha,
           W1k, b1k, W2k, b2k, W1l, b1l, W2l, b2l):
    del knn_alpha
    scale, flat, group = _meta_call(knn_tgt, knn_dist,
                                    W1k, b1k, W2k, b2k, W1l, b1l, W2l, b2l)
    out = _scale_call(nmt_prob, scale)
    ref = jax.new_ref(out.reshape(B * VOCAB))
    _sc_scatter_fn()(flat.reshape(NW * CH, 128),
                     group.reshape(NW * CH, 128), ref)
    return jax.freeze(ref).reshape(B, VOCAB)


# speedup vs baseline: 1.9572x; 1.9572x over previous
import functools

import jax
import jax.numpy as jnp
from jax import lax
from jax.experimental import pallas as pl
from jax.experimental.pallas import tpu as pltpu
from jax.experimental.pallas import tpu_sc as plsc

B = 1024
VOCAB = 100000
K = 64
RK = 7
TEMP = 10.0

BB = 128
NW = 32
RPW = B // NW
NG = RPW // 8
WSC = 1664
NQ = 12
EX_LO = 60 * WSC
SC_HI = EX_LO + 128
TAIL_BLK = SC_HI // 128

_f32 = jnp.float32


def _bf(x):
    return x.astype(jnp.bfloat16).astype(_f32)


def _mm_t(x, w):
    return lax.dot_general(_bf(x), _bf(w), (((1,), (1,)), ((), ())),
                           preferred_element_type=_f32)


def _meta_body(tgt_ref, dist_ref, w1k_ref, b1k_ref, w2k_ref, b2k_ref,
               w1l_ref, b1l_ref, w2l_ref, b2l_ref,
               scale_ref, group_ref, gfirst_ref):
    tgt = tgt_ref[...]
    dist = dist_ref[...]

    eqf = (tgt[:, :, None] == tgt[:, None, :]).astype(_f32)
    ii = lax.broadcasted_iota(jnp.int32, (K, K), 0)
    jj = lax.broadcasted_iota(jnp.int32, (K, K), 1)
    seen = jnp.sum(eqf * (jj < ii).astype(_f32)[None], axis=-1)
    novel = jnp.where((tgt != 0) & (seen == 0.0), 1.0, 0.0).astype(_f32)
    counts = jnp.dot(novel, (ii <= jj).astype(_f32),
                     preferred_element_type=_f32)

    net_in = jnp.concatenate([dist, counts], axis=-1)
    hk = jnp.tanh(_mm_t(net_in, w1k_ref[...]) + b1k_ref[...][None, :])
    lk = _mm_t(hk, w2k_ref[...]) + b2k_ref[...][None, :]
    mx = jnp.max(lk, axis=-1, keepdims=True)
    ek = jnp.exp(lk - mx)
    kp = ek / jnp.sum(ek, axis=-1, keepdims=True)

    hl = jnp.tanh(_mm_t(net_in, w1l_ref[...]) + b1l_ref[...][None, :])
    ll = jnp.sum(_bf(hl) * _bf(w2l_ref[...]), axis=-1,
                 keepdims=True) + b2l_ref[0]
    klam = jnp.minimum(jax.nn.sigmoid(ll), 0.99)

    ik = lax.broadcasted_iota(jnp.int32, (BB, K), 1)
    spare = jnp.zeros((BB, K), _f32)
    for r in range(RK):
        m = jnp.where(ik < (1 << r), 1.0, 1000.0).astype(_f32)
        logits = -(dist * m) / TEMP
        mxr = jnp.max(logits, axis=-1, keepdims=True)
        er = jnp.exp(logits - mxr)
        w = er / jnp.sum(er, axis=-1, keepdims=True)
        spare = spare + _bf(kp[:, r:r + 1]) * _bf(w)
    spare = klam * spare

    group = jnp.sum(eqf * spare[:, None, :], axis=-1)

    group_ref[...] = group
    gfirst_ref[...] = group * jnp.where(seen == 0.0, 1.0, 0.0)
    scale_ref[...] = (1.0 - klam) * jnp.ones((BB, 16), _f32)


def _meta_call(tgt, dist, w1k, b1k, w2k, b2k, w1l, b1l, w2l, b2l):
    full = lambda a: pl.BlockSpec(a.shape, lambda i: (0,) * a.ndim)
    return pl.pallas_call(
        _meta_body,
        grid=(B // BB,),
        in_specs=[
            pl.BlockSpec((BB, K), lambda i: (i, 0)),
            pl.BlockSpec((BB, K), lambda i: (i, 0)),
            full(w1k), full(b1k), full(w2k), full(b2k),
            full(w1l), full(b1l), full(w2l),
            pl.BlockSpec(memory_space=pltpu.SMEM),
        ],
        out_specs=[
            pl.BlockSpec((BB, 16), lambda i: (i, 0)),
            pl.BlockSpec((BB, K), lambda i: (i, 0)),
            pl.BlockSpec((BB, K), lambda i: (i, 0)),
        ],
        out_shape=[
            jax.ShapeDtypeStruct((B, 16), _f32),
            jax.ShapeDtypeStruct((B, K), _f32),
            jax.ShapeDtypeStruct((B, K), _f32),
        ],
    )(tgt, dist, w1k, b1k, w2k, b2k, w1l, b1l, w2l, b2l)


@functools.cache
def _sc_combine_fn():
    mesh = plsc.VectorSubcoreMesh(core_axis_name="c", subcore_axis_name="s")

    @functools.partial(
        pl.kernel,
        out_type=jax.ShapeDtypeStruct((B, VOCAB), _f32),
        mesh=mesh,
        compiler_params=pltpu.CompilerParams(needs_layout_passes=False),
        scratch_types=(
            [pltpu.VMEM((RPW, K), jnp.int32),
             pltpu.VMEM((RPW, K), _f32),
             pltpu.VMEM((RPW, 16), _f32)]
            + [pltpu.VMEM((8, WSC), _f32) for _ in range(5)]
            + [pltpu.VMEM((8, 128), _f32)]
            + [pltpu.SemaphoreType.DMA for _ in range(12)]
        ),
    )
    def _sc_combine(nmt, sc16, idx, val, out, idx_v, val_v, sc_v, *rest):
        bufs = rest[:5]
        bufe = rest[5]
        sins = rest[6:11]
        souts = rest[11:16]
        sine, soute = rest[16], rest[17]
        wid = lax.axis_index("s") * 2 + lax.axis_index("c")
        base = wid * RPW
        pltpu.sync_copy(idx.at[pl.ds(base, RPW)], idx_v)
        pltpu.sync_copy(val.at[pl.ds(base, RPW)], val_v)
        pltpu.sync_copy(sc16.at[pl.ds(base, RPW)], sc_v)

        zero16 = jnp.zeros((16,), jnp.int32)

        def scale_and_scatter(buf, g, lo, width):
            nvr = width // 16

            @pl.loop(0, nvr // 8)
            def _mul(i):
                o = i * 128
                for rr in range(8):
                    svec = sc_v[g * 8 + rr]
                    for u in range(8):
                        s = pl.ds(o + u * 16, 16)
                        buf[rr, s] = buf[rr, s] * svec

            if nvr % 8:
                o2 = (nvr // 8) * 128
                for rr in range(8):
                    svec = sc_v[g * 8 + rr]
                    for u in range(nvr % 8):
                        s = pl.ds(o2 + u * 16, 16)
                        buf[rr, s] = buf[rr, s] * svec

            for rr in range(8):
                parts = []
                for q in range(K // 16):
                    sg = pl.ds(q * 16, 16)
                    idx16 = idx_v[g * 8 + rr, sg]
                    val16 = val_v[g * 8 + rr, sg]
                    msk = (idx16 >= lo) & (idx16 < lo + width)
                    loc = idx16 - lo
                    cur = plsc.load_gather(buf, [zero16 + rr, loc], mask=msk)
                    parts.append((loc, cur + val16, msk))
                for loc, new16, msk in parts:
                    plsc.store_scatter(buf, [zero16 + rr, loc], new16,
                                       mask=msk)

        @pl.loop(0, NG)
        def _group(g):
            row8 = base + g * 8

            def in_sl(ch):
                return nmt.at[pl.ds(row8, 8), pl.ds(ch * WSC, WSC)]

            def out_sl(ch):
                return out.at[pl.ds(row8, 8), pl.ds(ch * WSC, WSC)]

            for h in range(5):
                pltpu.async_copy(in_sl(h), bufs[h], sins[h])

            @pl.loop(0, NQ)
            def _quint(q):
                for h in range(5):
                    ch = q * 5 + h
                    buf, sin, sout = bufs[h], sins[h], souts[h]
                    pltpu.make_async_copy(in_sl(ch), buf, sin).wait()
                    scale_and_scatter(buf, g, ch * WSC, WSC)
                    pltpu.async_copy(buf, out_sl(ch), sout)
                    if h >= 1:
                        pltpu.make_async_copy(bufs[h - 1], out_sl(ch - 1),
                                              souts[h - 1]).wait()

                        @pl.when(ch + 4 < 60)
                        def _():
                            pltpu.async_copy(in_sl(ch + 4), bufs[h - 1],
                                             sins[h - 1])
                    else:
                        @pl.when(q > 0)
                        def _():
                            pltpu.make_async_copy(bufs[4], out_sl(ch - 1),
                                                  souts[4]).wait()
                            pltpu.async_copy(in_sl(ch + 4), bufs[4], sins[4])

            pltpu.make_async_copy(bufs[4], out_sl(59), souts[4]).wait()

            esl_in = nmt.at[pl.ds(row8, 8), pl.ds(EX_LO, 128)]
            esl_out = out.at[pl.ds(row8, 8), pl.ds(EX_LO, 128)]
            pltpu.async_copy(esl_in, bufe, sine).wait()
            scale_and_scatter(bufe, g, EX_LO, 128)
            pltpu.async_copy(bufe, esl_out, soute).wait()

    return _sc_combine


def _tail_body(out_in_ref, nmt_ref, sc_ref, tgt_ref, gf_ref, out_ref):
    del out_in_ref
    scale = sc_ref[...][:, 0:1]
    acc = nmt_ref[...] * scale
    cols = TAIL_BLK * 128 + lax.broadcasted_iota(
        jnp.int32, (out_ref.shape[0], 128), 1)
    tgt = tgt_ref[...]
    gf = gf_ref[...]
    for k in range(K):
        acc = acc + jnp.where(tgt[:, k:k + 1] == cols, gf[:, k:k + 1], 0.0)
    out_ref[...] = acc


def _tail_call(out_sc, nmt_prob, scale16, tgt, gfirst):
    BBt = 256
    return pl.pallas_call(
        _tail_body,
        grid=(B // BBt,),
        in_specs=[
            pl.BlockSpec((BBt, 128), lambda i: (i, TAIL_BLK)),
            pl.BlockSpec((BBt, 128), lambda i: (i, TAIL_BLK)),
            pl.BlockSpec((BBt, 16), lambda i: (i, 0)),
            pl.BlockSpec((BBt, K), lambda i: (i, 0)),
            pl.BlockSpec((BBt, K), lambda i: (i, 0)),
        ],
        out_specs=pl.BlockSpec((BBt, 128), lambda i: (i, TAIL_BLK)),
        out_shape=jax.ShapeDtypeStruct((B, VOCAB), _f32),
        input_output_aliases={0: 0},
    )(out_sc, nmt_prob, scale16, tgt, gfirst)


def kernel(nmt_prob, knn_tgt, knn_dist, knn_alpha,
           W1k, b1k, W2k, b2k, W1l, b1l, W2l, b2l):
    del knn_alpha
    scale16, group, gfirst = _meta_call(knn_tgt, knn_dist,
                                        W1k, b1k, W2k, b2k, W1l, b1l, W2l,
                                        b2l)
    out_sc = _sc_combine_fn()(nmt_prob, scale16, knn_tgt, group)
    return _tail_call(out_sc, nmt_prob, scale16, knn_tgt, gfirst)


# speedup vs baseline: 3.3880x; 1.7310x over previous
import functools

import jax
import jax.numpy as jnp
from jax import lax
from jax.experimental import pallas as pl
from jax.experimental.pallas import tpu as pltpu
from jax.experimental.pallas import tpu_sc as plsc

B = 1024
VOCAB = 100000
K = 64
RK = 7
TEMP = 10.0

BB = 128
NW = 32
VT = VOCAB // 8
NTW = 390
NEX = VT - NW * NTW
NSEG = 13
CAPG = 8192
CAPS = 2048

_f32 = jnp.float32


def _bf(x):
    return x.astype(jnp.bfloat16).astype(_f32)


def _mm_t(x, w):
    return lax.dot_general(_bf(x), _bf(w), (((1,), (1,)), ((), ())),
                           preferred_element_type=_f32)


def _meta_body(tgt_ref, dist_ref, w1k_ref, b1k_ref, w2k_ref, b2k_ref,
               w1l_ref, b1l_ref, w2l_ref, b2l_ref,
               scale_ref, group_ref):
    tgt = tgt_ref[...]
    dist = dist_ref[...]

    eqf = (tgt[:, :, None] == tgt[:, None, :]).astype(_f32)
    ii = lax.broadcasted_iota(jnp.int32, (K, K), 0)
    jj = lax.broadcasted_iota(jnp.int32, (K, K), 1)
    seen = jnp.sum(eqf * (jj < ii).astype(_f32)[None], axis=-1)
    novel = jnp.where((tgt != 0) & (seen == 0.0), 1.0, 0.0).astype(_f32)
    counts = jnp.dot(novel, (ii <= jj).astype(_f32),
                     preferred_element_type=_f32)

    net_in = jnp.concatenate([dist, counts], axis=-1)
    hk = jnp.tanh(_mm_t(net_in, w1k_ref[...]) + b1k_ref[...][None, :])
    lk = _mm_t(hk, w2k_ref[...]) + b2k_ref[...][None, :]
    mx = jnp.max(lk, axis=-1, keepdims=True)
    ek = jnp.exp(lk - mx)
    kp = ek / jnp.sum(ek, axis=-1, keepdims=True)

    hl = jnp.tanh(_mm_t(net_in, w1l_ref[...]) + b1l_ref[...][None, :])
    ll = jnp.sum(_bf(hl) * _bf(w2l_ref[...]), axis=-1,
                 keepdims=True) + b2l_ref[0]
    klam = jnp.minimum(jax.nn.sigmoid(ll), 0.99)

    ik = lax.broadcasted_iota(jnp.int32, (BB, K), 1)
    spare = jnp.zeros((BB, K), _f32)
    for r in range(RK):
        m = jnp.where(ik < (1 << r), 1.0, 1000.0).astype(_f32)
        logits = -(dist * m) / TEMP
        mxr = jnp.max(logits, axis=-1, keepdims=True)
        er = jnp.exp(logits - mxr)
        w = er / jnp.sum(er, axis=-1, keepdims=True)
        spare = spare + _bf(kp[:, r:r + 1]) * _bf(w)
    spare = klam * spare

    group = jnp.sum(eqf * spare[:, None, :], axis=-1)

    group_ref[...] = group
    scale_ref[...] = (1.0 - klam) * jnp.ones((BB, 16), _f32)


def _meta_call(tgt, dist, w1k, b1k, w2k, b2k, w1l, b1l, w2l, b2l):
    full = lambda a: pl.BlockSpec(a.shape, lambda i: (0,) * a.ndim)
    return pl.pallas_call(
        _meta_body,
        grid=(B // BB,),
        in_specs=[
            pl.BlockSpec((BB, K), lambda i: (i, 0)),
            pl.BlockSpec((BB, K), lambda i: (i, 0)),
            full(w1k), full(b1k), full(w2k), full(b2k),
            full(w1l), full(b1l), full(w2l),
            pl.BlockSpec(memory_space=pltpu.SMEM),
        ],
        out_specs=[
            pl.BlockSpec((BB, 16), lambda i: (i, 0)),
            pl.BlockSpec((BB, K), lambda i: (i, 0)),
        ],
        out_shape=[
            jax.ShapeDtypeStruct((B, 16), _f32),
            jax.ShapeDtypeStruct((B, K), _f32),
        ],
    )(tgt, dist, w1k, b1k, w2k, b2k, w1l, b1l, w2l, b2l)


@functools.cache
def _sc_combine_fn():
    mesh = plsc.VectorSubcoreMesh(core_axis_name="c", subcore_axis_name="s")

    @functools.partial(
        pl.kernel,
        out_type=jax.ShapeDtypeStruct((VOCAB, B), _f32),
        mesh=mesh,
        compiler_params=pltpu.CompilerParams(needs_layout_passes=False),
        scratch_types=(
            [pltpu.VMEM((128, K), jnp.int32),
             pltpu.VMEM((128, K), _f32),
             pltpu.VMEM((CAPG + 16,), jnp.int32),
             pltpu.VMEM((CAPG + 16,), jnp.int32),
             pltpu.VMEM((CAPG + 16,), _f32),
             pltpu.VMEM((CAPS + 16,), jnp.int32),
             pltpu.VMEM((CAPS + 16,), jnp.int32),
             pltpu.VMEM((CAPS + 16,), _f32),
             pltpu.VMEM((CAPS + 16,), _f32),
             pltpu.VMEM((8, B), _f32)]
            + [pltpu.VMEM((8, B), _f32) for _ in range(6)]
            + [pltpu.SemaphoreType.DMA for _ in range(12)]
        ),
    )
    def _sc_combine(nmt, sc8, tgt, val, out, t_st, v_st, tl, bl, vl,
                    stl, sbl, svl, stage, sc_v, *rest):
        bufs = rest[:5]
        bufe = rest[5]
        sins = rest[6:11]
        souts = rest[11:16]
        sine, soute = rest[16], rest[17]
        wid = lax.axis_index("s") * 2 + lax.axis_index("c")
        ts = wid * NTW
        main_lo = ts * 8
        main_hi = main_lo + NTW * 8
        ex_lo = jnp.where(wid < NEX, (NW * NTW + wid) * 8, 2 * VOCAB)

        pltpu.sync_copy(sc8, sc_v)

        off = jnp.int32(0)
        for p in range(8):
            pltpu.sync_copy(tgt.at[pl.ds(p * 128, 128)], t_st)
            pltpu.sync_copy(val.at[pl.ds(p * 128, 128)], v_st)

            @pl.loop(0, 128, init_carry=off)
            def _scan(r, o):
                for g in range(K // 16):
                    sg = pl.ds(g * 16, 16)
                    t16 = t_st[r, sg]
                    v16 = v_st[r, sg]
                    b16 = jnp.zeros((16,), jnp.int32) + (p * 128 + r)
                    m = ((t16 >= main_lo) & (t16 < main_hi)) | (
                        (t16 >= ex_lo) & (t16 < ex_lo + 8))
                    o = jnp.minimum(o, CAPG)
                    plsc.store_compressed(tl.at[pl.ds(o, 16)], t16, mask=m)
                    plsc.store_compressed(bl.at[pl.ds(o, 16)], b16, mask=m)
                    plsc.store_compressed(vl.at[pl.ds(o, 16)], v16, mask=m)
                    o = o + plsc.all_reduce_population_count(m)[0]
                return o

            off = _scan
        total = jnp.minimum(off, CAPG)

        def apply_updates(buf, base_t, ngrp, t_l, b_l, v_l):
            @pl.loop(0, ngrp)
            def _ga(g):
                sg = pl.ds(g * 16, 16)
                t16 = t_l[sg]
                b16 = b_l[sg]
                m = (t16 >= base_t) & (t16 < base_t + 8)
                cur = plsc.load_gather(buf, [t16 - base_t, b16], mask=m)
                stage[sg] = cur + v_l[sg]

            @pl.loop(0, ngrp)
            def _sc(g):
                sg = pl.ds(g * 16, 16)
                t16 = t_l[sg]
                b16 = b_l[sg]
                m = (t16 >= base_t) & (t16 < base_t + 8)
                plsc.store_scatter(buf, [t16 - base_t, b16], stage[sg],
                                   mask=m)

        def multiply(buf):
            @pl.loop(0, B // 16)
            def _mul(u):
                s = pl.ds(u * 16, 16)
                svec = sc_v[0, s]
                for rr in range(8):
                    buf[rr, s] = buf[rr, s] * svec

        def in_sl(ch):
            return nmt.at[pl.ds((ts + ch) * 8, 8)]

        def out_sl(ch):
            return out.at[pl.ds((ts + ch) * 8, 8)]

        for h in range(5):
            pltpu.async_copy(in_sl(h), bufs[h], sins[h])

        @pl.loop(0, NSEG)
        def _seg(s):
            seg_lo = main_lo + s * 240
            seg_hi = seg_lo + 240

            @pl.loop(0, (total + 15) // 16, init_carry=jnp.int32(0))
            def _filt(g, so):
                sg = pl.ds(g * 16, 16)
                t16 = tl[sg]
                m = (t16 >= seg_lo) & (t16 < seg_hi)
                so = jnp.minimum(so, CAPS)
                plsc.store_compressed(stl.at[pl.ds(so, 16)], t16, mask=m)
                plsc.store_compressed(sbl.at[pl.ds(so, 16)], bl[sg], mask=m)
                plsc.store_compressed(svl.at[pl.ds(so, 16)], vl[sg], mask=m)
                return so + plsc.all_reduce_population_count(m)[0]

            ngrp = (jnp.minimum(_filt, CAPS) + 15) // 16

            @pl.loop(0, 6)
            def _quint(q):
                Q = s * 6 + q
                for h in range(5):
                    ch = Q * 5 + h
                    buf, sin, sout = bufs[h], sins[h], souts[h]
                    pltpu.make_async_copy(in_sl(ch), buf, sin).wait()
                    multiply(buf)
                    apply_updates(buf, (ts + ch) * 8, ngrp, stl, sbl, svl)
                    pltpu.async_copy(buf, out_sl(ch), sout)
                    if h >= 1:
                        pltpu.make_async_copy(bufs[h - 1], out_sl(ch - 1),
                                              souts[h - 1]).wait()

                        @pl.when(ch + 4 < NTW)
                        def _():
                            pltpu.async_copy(in_sl(ch + 4), bufs[h - 1],
                                             sins[h - 1])
                    else:
                        @pl.when(Q > 0)
                        def _():
                            pltpu.make_async_copy(bufs[4], out_sl(ch - 1),
                                                  souts[4]).wait()
                            pltpu.async_copy(in_sl(ch + 4), bufs[4], sins[4])

        pltpu.make_async_copy(bufs[4], out_sl(NTW - 1), souts[4]).wait()

        @pl.when(wid < NEX)
        def _extra():
            tr = NW * NTW + wid
            esl_in = nmt.at[pl.ds(tr * 8, 8)]
            esl_out = out.at[pl.ds(tr * 8, 8)]
            pltpu.async_copy(esl_in, bufe, sine).wait()
            multiply(bufe)
            apply_updates(bufe, tr * 8, (total + 15) // 16, tl, bl, vl)
            pltpu.async_copy(bufe, esl_out, soute).wait()

    return _sc_combine


def kernel(nmt_prob, knn_tgt, knn_dist, knn_alpha,
           W1k, b1k, W2k, b2k, W1l, b1l, W2l, b2l):
    del knn_alpha
    scale16, group = _meta_call(knn_tgt, knn_dist,
                                W1k, b1k, W2k, b2k, W1l, b1l, W2l, b2l)
    scale8 = jnp.broadcast_to(scale16[:, 0][None, :], (8, B))
    out_t = _sc_combine_fn()(nmt_prob.T, scale8, knn_tgt, group)
    return out_t.T


# speedup vs baseline: 5.0826x; 1.5002x over previous
import functools

import jax
import jax.numpy as jnp
from jax import lax
from jax.experimental import pallas as pl
from jax.experimental.pallas import tpu as pltpu
from jax.experimental.pallas import tpu_sc as plsc

B = 1024
VOCAB = 100000
K = 64
RK = 7
TEMP = 10.0

BB = 128
NW = 32
VT = VOCAB // 8
NTW = 390
NEX = VT - NW * NTW
NSEG = 13
CAPG = 8192
CAPS = 2048

_f32 = jnp.float32


def _bf(x):
    return x.astype(jnp.bfloat16).astype(_f32)


def _mm_t(x, w):
    return lax.dot_general(_bf(x), _bf(w), (((1,), (1,)), ((), ())),
                           preferred_element_type=_f32)


def _meta_body(tgt_ref, dist_ref, w1k_ref, b1k_ref, w2k_ref, b2k_ref,
               w1l_ref, b1l_ref, w2l_ref, b2l_ref,
               scale_ref, group_ref):
    tgt = tgt_ref[...]
    dist = dist_ref[...]

    eqf = (tgt[:, :, None] == tgt[:, None, :]).astype(_f32)
    ii = lax.broadcasted_iota(jnp.int32, (K, K), 0)
    jj = lax.broadcasted_iota(jnp.int32, (K, K), 1)
    seen = jnp.sum(eqf * (jj < ii).astype(_f32)[None], axis=-1)
    novel = jnp.where((tgt != 0) & (seen == 0.0), 1.0, 0.0).astype(_f32)
    counts = jnp.dot(novel, (ii <= jj).astype(_f32),
                     preferred_element_type=_f32)

    net_in = jnp.concatenate([dist, counts], axis=-1)
    hk = jnp.tanh(_mm_t(net_in, w1k_ref[...]) + b1k_ref[...][None, :])
    lk = _mm_t(hk, w2k_ref[...]) + b2k_ref[...][None, :]
    mx = jnp.max(lk, axis=-1, keepdims=True)
    ek = jnp.exp(lk - mx)
    kp = ek / jnp.sum(ek, axis=-1, keepdims=True)

    hl = jnp.tanh(_mm_t(net_in, w1l_ref[...]) + b1l_ref[...][None, :])
    ll = jnp.sum(_bf(hl) * _bf(w2l_ref[...]), axis=-1,
                 keepdims=True) + b2l_ref[0]
    klam = jnp.minimum(jax.nn.sigmoid(ll), 0.99)

    ik = lax.broadcasted_iota(jnp.int32, (BB, K), 1)
    spare = jnp.zeros((BB, K), _f32)
    for r in range(RK):
        m = jnp.where(ik < (1 << r), 1.0, 1000.0).astype(_f32)
        logits = -(dist * m) / TEMP
        mxr = jnp.max(logits, axis=-1, keepdims=True)
        er = jnp.exp(logits - mxr)
        w = er / jnp.sum(er, axis=-1, keepdims=True)
        spare = spare + _bf(kp[:, r:r + 1]) * _bf(w)
    spare = klam * spare

    group = jnp.sum(eqf * spare[:, None, :], axis=-1)

    group_ref[...] = group
    scale_ref[...] = (1.0 - klam) * jnp.ones((BB, 16), _f32)


def _meta_call(tgt, dist, w1k, b1k, w2k, b2k, w1l, b1l, w2l, b2l):
    full = lambda a: pl.BlockSpec(a.shape, lambda i: (0,) * a.ndim)
    return pl.pallas_call(
        _meta_body,
        grid=(B // BB,),
        in_specs=[
            pl.BlockSpec((BB, K), lambda i: (i, 0)),
            pl.BlockSpec((BB, K), lambda i: (i, 0)),
            full(w1k), full(b1k), full(w2k), full(b2k),
            full(w1l), full(b1l), full(w2l),
            pl.BlockSpec(memory_space=pltpu.SMEM),
        ],
        out_specs=[
            pl.BlockSpec((BB, 16), lambda i: (i, 0)),
            pl.BlockSpec((BB, K), lambda i: (i, 0)),
        ],
        out_shape=[
            jax.ShapeDtypeStruct((B, 16), _f32),
            jax.ShapeDtypeStruct((B, K), _f32),
        ],
    )(tgt, dist, w1k, b1k, w2k, b2k, w1l, b1l, w2l, b2l)


@functools.cache
def _sc_combine_fn():
    mesh = plsc.VectorSubcoreMesh(core_axis_name="c", subcore_axis_name="s")

    @functools.partial(
        pl.kernel,
        out_type=jax.ShapeDtypeStruct((VOCAB, B), _f32),
        mesh=mesh,
        compiler_params=pltpu.CompilerParams(needs_layout_passes=False),
        scratch_types=(
            [pltpu.VMEM((128, K), jnp.int32),
             pltpu.VMEM((128, K), _f32),
             pltpu.VMEM((CAPG + 16,), jnp.int32),
             pltpu.VMEM((CAPG + 16,), jnp.int32),
             pltpu.VMEM((CAPG + 16,), _f32),
             pltpu.VMEM((CAPS + 16,), jnp.int32),
             pltpu.VMEM((CAPS + 16,), jnp.int32),
             pltpu.VMEM((CAPS + 16,), _f32),
             pltpu.VMEM((CAPS + 16,), _f32),
             pltpu.VMEM((8, B), _f32)]
            + [pltpu.VMEM((8, B), _f32) for _ in range(6)]
            + [pltpu.SemaphoreType.DMA for _ in range(12)]
        ),
    )
    def _sc_combine(nmt, sc8, tgt, val, out, t_st, v_st, tl, bl, vl,
                    stl, sbl, svl, stage, sc_v, *rest):
        bufs = rest[:5]
        bufe = rest[5]
        sins = rest[6:11]
        souts = rest[11:16]
        sine, soute = rest[16], rest[17]
        wid = lax.axis_index("s") * 2 + lax.axis_index("c")
        ts = wid * NTW
        main_lo = ts * 8
        main_hi = main_lo + NTW * 8
        ex_lo = jnp.where(wid < NEX, (NW * NTW + wid) * 8, 2 * VOCAB)

        pltpu.sync_copy(sc8, sc_v)

        off = jnp.int32(0)
        for p in range(8):
            pltpu.sync_copy(tgt.at[pl.ds(p * 128, 128)], t_st)
            pltpu.sync_copy(val.at[pl.ds(p * 128, 128)], v_st)

            @pl.loop(0, 128, init_carry=off)
            def _scan(r, o):
                for g in range(K // 16):
                    sg = pl.ds(g * 16, 16)
                    t16 = t_st[r, sg]
                    v16 = v_st[r, sg]
                    b16 = jnp.zeros((16,), jnp.int32) + (p * 128 + r)
                    m = ((t16 >= main_lo) & (t16 < main_hi)) | (
                        (t16 >= ex_lo) & (t16 < ex_lo + 8))
                    o = jnp.minimum(o, CAPG)
                    plsc.store_compressed(tl.at[pl.ds(o, 16)], t16, mask=m)
                    plsc.store_compressed(bl.at[pl.ds(o, 16)], b16, mask=m)
                    plsc.store_compressed(vl.at[pl.ds(o, 16)], v16, mask=m)
                    o = o + plsc.all_reduce_population_count(m)[0]
                return o

            off = _scan
        total = jnp.minimum(off, CAPG)

        def apply_updates(buf, base_t, ngrp, t_l, b_l, v_l):
            @pl.loop(0, ngrp)
            def _ga(g):
                sg = pl.ds(g * 16, 16)
                t16 = t_l[sg]
                b16 = b_l[sg]
                m = (t16 >= base_t) & (t16 < base_t + 8)
                cur = plsc.load_gather(buf, [t16 - base_t, b16], mask=m)
                stage[sg] = cur + v_l[sg]

            @pl.loop(0, ngrp)
            def _sc(g):
                sg = pl.ds(g * 16, 16)
                t16 = t_l[sg]
                b16 = b_l[sg]
                m = (t16 >= base_t) & (t16 < base_t + 8)
                plsc.store_scatter(buf, [t16 - base_t, b16], stage[sg],
                                   mask=m)

        def multiply(buf):
            @pl.loop(0, B // 128)
            def _mul(uo):
                base = uo * 128
                for ui in range(8):
                    s = pl.ds(base + ui * 16, 16)
                    svec = sc_v[0, s]
                    for rr in range(8):
                        buf[rr, s] = buf[rr, s] * svec

        def in_sl(ch):
            return nmt.at[pl.ds((ts + ch) * 8, 8)]

        def out_sl(ch):
            return out.at[pl.ds((ts + ch) * 8, 8)]

        for h in range(5):
            pltpu.async_copy(in_sl(h), bufs[h], sins[h])

        @pl.loop(0, NSEG)
        def _seg(s):
            seg_lo = main_lo + s * 240
            seg_hi = seg_lo + 240

            @pl.loop(0, (total + 15) // 16, init_carry=jnp.int32(0))
            def _filt(g, so):
                sg = pl.ds(g * 16, 16)
                t16 = tl[sg]
                m = (t16 >= seg_lo) & (t16 < seg_hi)
                so = jnp.minimum(so, CAPS)
                plsc.store_compressed(stl.at[pl.ds(so, 16)], t16, mask=m)
                plsc.store_compressed(sbl.at[pl.ds(so, 16)], bl[sg], mask=m)
                plsc.store_compressed(svl.at[pl.ds(so, 16)], vl[sg], mask=m)
                return so + plsc.all_reduce_population_count(m)[0]

            ngrp = (jnp.minimum(_filt, CAPS) + 15) // 16

            @pl.loop(0, 6)
            def _quint(q):
                Q = s * 6 + q
                for h in range(5):
                    ch = Q * 5 + h
                    buf, sin, sout = bufs[h], sins[h], souts[h]
                    pltpu.make_async_copy(in_sl(ch), buf, sin).wait()
                    multiply(buf)
                    apply_updates(buf, (ts + ch) * 8, ngrp, stl, sbl, svl)
                    pltpu.async_copy(buf, out_sl(ch), sout)
                    if h >= 1:
                        pltpu.make_async_copy(bufs[h - 1], out_sl(ch - 1),
                                              souts[h - 1]).wait()

                        @pl.when(ch + 4 < NTW)
                        def _():
                            pltpu.async_copy(in_sl(ch + 4), bufs[h - 1],
                                             sins[h - 1])
                    else:
                        @pl.when(Q > 0)
                        def _():
                            pltpu.make_async_copy(bufs[4], out_sl(ch - 1),
                                                  souts[4]).wait()
                            pltpu.async_copy(in_sl(ch + 4), bufs[4], sins[4])

        pltpu.make_async_copy(bufs[4], out_sl(NTW - 1), souts[4]).wait()

        @pl.when(wid < NEX)
        def _extra():
            tr = NW * NTW + wid
            esl_in = nmt.at[pl.ds(tr * 8, 8)]
            esl_out = out.at[pl.ds(tr * 8, 8)]
            pltpu.async_copy(esl_in, bufe, sine).wait()
            multiply(bufe)
            apply_updates(bufe, tr * 8, (total + 15) // 16, tl, bl, vl)
            pltpu.async_copy(bufe, esl_out, soute).wait()

    return _sc_combine


def kernel(nmt_prob, knn_tgt, knn_dist, knn_alpha,
           W1k, b1k, W2k, b2k, W1l, b1l, W2l, b2l):
    del knn_alpha
    scale16, group = _meta_call(knn_tgt, knn_dist,
                                W1k, b1k, W2k, b2k, W1l, b1l, W2l, b2l)
    scale8 = jnp.broadcast_to(scale16[:, 0][None, :], (8, B))
    out_t = _sc_combine_fn()(nmt_prob.T, scale8, knn_tgt, group)
    return out_t.T
